# baseline (speedup 1.0000x reference)
from math import exp, log

import numpy as np
import jax
import jax.numpy as jnp
from jax import lax
from jax.experimental import pallas as pl
from jax.experimental.pallas import tpu as pltpu
from jax.experimental.pallas import tpu_sc as plsc

N_LEVELS = 16
F_PER_LEVEL = 2
LOG2_T = 19
T = 1 << LOG2_T
BASE_RES = 16
MAX_RES = 2048
SCALE = exp((log(MAX_RES) - log(BASE_RES)) / (N_LEVELS - 1))
RES = [int(np.floor(BASE_RES * (SCALE ** l))) for l in range(N_LEVELS)]
DENSE = [(r + 1) ** 3 <= T for r in RES]
P1 = np.int32(-1640531535)
P2 = np.int32(805459861)
HMASK = np.int32(T - 1)
CORNERS = [(i, j, k) for i in (0, 1) for j in (0, 1) for k in (0, 1)]

N_POINTS = 524288
NC, NS = 2, 16
NW = NC * NS
NPT = N_POINTS // NW
C = 512
G = C // 16
NCH = NPT // C
NIDX = 16 * C


def _body(x_hbm, tab_hbm, out_hbm, xbuf, fracbuf, idxbuf, gath, outc, sem):
    cid = lax.axis_index("c")
    sid = lax.axis_index("s")
    wid = sid * NC + cid
    lane = lax.iota(jnp.int32, 16)
    lane3 = lane * 3
    lane2 = lane * 2

    def chunk_body(ch, carry):
        base = wid * NPT + ch * C
        pltpu.sync_copy(x_hbm.at[pl.ds(base * 3, C * 3)], xbuf)
        for l in range(N_LEVELS):
            res = RES[l]
            resf = np.float32(res)

            def p1(g, c1):
                b16 = g * 16
                xi = b16 * 3 + lane3
                px = plsc.load_gather(xbuf, [xi])
                py = plsc.load_gather(xbuf, [xi + 1])
                pz = plsc.load_gather(xbuf, [xi + 2])
                posx = px * resf
                posy = py * resf
                posz = pz * resf
                ix = posx.astype(jnp.int32)
                iy = posy.astype(jnp.int32)
                iz = posz.astype(jnp.int32)
                fracbuf[0, pl.ds(b16, 16)] = posx - ix.astype(jnp.float32)
                fracbuf[1, pl.ds(b16, 16)] = posy - iy.astype(jnp.float32)
                fracbuf[2, pl.ds(b16, 16)] = posz - iz.astype(jnp.float32)
                posbase = g * 32 + lane2
                for ci, (i, j, k) in enumerate(CORNERS):
                    cx = ix + i if i else ix
                    cy = iy + j if j else iy
                    cz = iz + k if k else iz
                    if DENSE[l]:
                        s = np.int32(res + 1)
                        idx = cx + cy * s + cz * np.int32((res + 1) * (res + 1))
                    else:
                        idx = (cx ^ (cy * P1) ^ (cz * P2)) & HMASK
                    ev = (idx + np.int32(l * T)) * 2
                    posv = posbase + np.int32(ci * 2 * C)
                    plsc.store_scatter(idxbuf, [posv], ev)
                    plsc.store_scatter(idxbuf, [posv + 1], ev + 1)
                return c1

            lax.fori_loop(0, G, p1, 0, unroll=False)

            pltpu.async_copy(tab_hbm.at[idxbuf], gath, sem).wait()

            col2l0 = jnp.full((16,), 2 * l, jnp.int32)
            col2l1 = jnp.full((16,), 2 * l + 1, jnp.int32)

            def p2(g, c2):
                b16 = g * 16
                fx = fracbuf[0, pl.ds(b16, 16)]
                fy = fracbuf[1, pl.ds(b16, 16)]
                fz = fracbuf[2, pl.ds(b16, 16)]
                gx = 1.0 - fx
                gy = 1.0 - fy
                gz = 1.0 - fz
                wyz = (gy * gz, gy * fz, fy * gz, fy * fz)
                posbase = g * 32 + lane2
                acc0 = jnp.zeros((16,), jnp.float32)
                acc1 = jnp.zeros((16,), jnp.float32)
                for ci, (i, j, k) in enumerate(CORNERS):
                    posv = posbase + np.int32(ci * 2 * C)
                    f0 = plsc.load_gather(gath, [posv])
                    f1 = plsc.load_gather(gath, [posv + 1])
                    w = (fx if i else gx) * wyz[2 * j + k]
                    acc0 = acc0 + w * f0
                    acc1 = acc1 + w * f1
                rowv = b16 + lane
                plsc.store_scatter(outc, [rowv, col2l0], acc0)
                plsc.store_scatter(outc, [rowv, col2l1], acc1)
                return c2

            lax.fori_loop(0, G, p2, 0, unroll=False)
        pltpu.sync_copy(outc, out_hbm.at[pl.ds(base, C)])
        return carry

    lax.fori_loop(0, NCH, chunk_body, 0, unroll=False)


_mesh = plsc.VectorSubcoreMesh(
    core_axis_name="c", subcore_axis_name="s", num_cores=2, num_subcores=16
)

_call = pl.kernel(
    _body,
    out_type=jax.ShapeDtypeStruct((N_POINTS, N_LEVELS * F_PER_LEVEL), jnp.float32),
    mesh=_mesh,
    scratch_types=[
        pltpu.VMEM((3 * C,), jnp.float32),
        pltpu.VMEM((3, C), jnp.float32),
        pltpu.VMEM((NIDX,), jnp.int32),
        pltpu.VMEM((NIDX,), jnp.float32),
        pltpu.VMEM((C, N_LEVELS * F_PER_LEVEL), jnp.float32),
        pltpu.SemaphoreType.DMA,
    ],
    compiler_params=pltpu.CompilerParams(
        needs_layout_passes=False, use_tc_tiling_on_sc=False
    ),
)


def kernel(x, table):
    xf = x.reshape(-1)
    tf = table.reshape(-1)
    return _call(xf, tf)


# speedup vs baseline: 4.1125x; 4.1125x over previous
from math import exp, log

import numpy as np
import jax
import jax.numpy as jnp
from jax import lax
from jax.experimental import pallas as pl
from jax.experimental.pallas import tpu as pltpu
from jax.experimental.pallas import tpu_sc as plsc

N_LEVELS = 16
F_PER_LEVEL = 2
LOG2_T = 19
T = 1 << LOG2_T
BASE_RES = 16
MAX_RES = 2048
SCALE = exp((log(MAX_RES) - log(BASE_RES)) / (N_LEVELS - 1))
RES = [int(np.floor(BASE_RES * (SCALE ** l))) for l in range(N_LEVELS)]
DENSE = [(r + 1) ** 3 <= T for r in RES]
P1 = np.int32(-1640531535)
P2 = np.int32(805459861)
HMASK = np.int32(T - 1)
CORNERS = [(i, j, k) for i in (0, 1) for j in (0, 1) for k in (0, 1)]

N_POINTS = 524288
NC, NS = 2, 16
NW = NC * NS
NPT = N_POINTS // NW
C = 512
G = C // 16
NCH = NPT // C
NIDX = 8 * C
HI16 = np.int32(-65536)


def _body(x_hbm, tab_hbm, out_hbm, xbuf, fracbuf, idxbuf, gath, outc, sem):
    cid = lax.axis_index("c")
    sid = lax.axis_index("s")
    wid = sid * NC + cid
    lane = lax.iota(jnp.int32, 16)
    lane3 = lane * 3

    def chunk_body(ch, carry):
        base = wid * NPT + ch * C
        pltpu.sync_copy(x_hbm.at[pl.ds(base * 3, C * 3)], xbuf)
        for l in range(N_LEVELS):
            res = RES[l]
            resf = np.float32(res)

            def p1(g, c1):
                b16 = g * 16
                xi = b16 * 3 + lane3
                px = plsc.load_gather(xbuf, [xi])
                py = plsc.load_gather(xbuf, [xi + 1])
                pz = plsc.load_gather(xbuf, [xi + 2])
                posx = px * resf
                posy = py * resf
                posz = pz * resf
                ix = posx.astype(jnp.int32)
                iy = posy.astype(jnp.int32)
                iz = posz.astype(jnp.int32)
                fracbuf[0, pl.ds(b16, 16)] = posx - ix.astype(jnp.float32)
                fracbuf[1, pl.ds(b16, 16)] = posy - iy.astype(jnp.float32)
                fracbuf[2, pl.ds(b16, 16)] = posz - iz.astype(jnp.float32)
                for ci, (i, j, k) in enumerate(CORNERS):
                    cx = ix + i if i else ix
                    cy = iy + j if j else iy
                    cz = iz + k if k else iz
                    if DENSE[l]:
                        s = np.int32(res + 1)
                        idx = cx + cy * s + cz * np.int32((res + 1) * (res + 1))
                    else:
                        idx = (cx ^ (cy * P1) ^ (cz * P2)) & HMASK
                    idxbuf[pl.ds(ci * C + b16, 16)] = idx + np.int32(l * T)
                return c1

            lax.fori_loop(0, G, p1, 0, unroll=False)

            pltpu.async_copy(tab_hbm.at[idxbuf], gath, sem).wait()

            def p2(g, c2):
                b16 = g * 16
                fx = fracbuf[0, pl.ds(b16, 16)]
                fy = fracbuf[1, pl.ds(b16, 16)]
                fz = fracbuf[2, pl.ds(b16, 16)]
                gx = 1.0 - fx
                gy = 1.0 - fy
                gz = 1.0 - fz
                wyz = (gy * gz, gy * fz, fy * gz, fy * fz)
                acc0 = jnp.zeros((16,), jnp.float32)
                acc1 = jnp.zeros((16,), jnp.float32)
                for ci, (i, j, k) in enumerate(CORNERS):
                    v = gath[pl.ds(ci * C + b16, 16)]
                    f0 = plsc.bitcast(lax.shift_left(v, 16), jnp.float32)
                    f1 = plsc.bitcast(v & HI16, jnp.float32)
                    w = (fx if i else gx) * wyz[2 * j + k]
                    acc0 = acc0 + w * f0
                    acc1 = acc1 + w * f1
                outc[2 * l, pl.ds(b16, 16)] = acc0
                outc[2 * l + 1, pl.ds(b16, 16)] = acc1
                return c2

            lax.fori_loop(0, G, p2, 0, unroll=False)
        pltpu.sync_copy(outc, out_hbm.at[:, pl.ds(base, C)])
        return carry

    lax.fori_loop(0, NCH, chunk_body, 0, unroll=False)


_mesh = plsc.VectorSubcoreMesh(
    core_axis_name="c", subcore_axis_name="s", num_cores=2, num_subcores=16
)

_call = pl.kernel(
    _body,
    out_type=jax.ShapeDtypeStruct((N_LEVELS * F_PER_LEVEL, N_POINTS), jnp.float32),
    mesh=_mesh,
    scratch_types=[
        pltpu.VMEM((3 * C,), jnp.float32),
        pltpu.VMEM((3, C), jnp.float32),
        pltpu.VMEM((NIDX,), jnp.int32),
        pltpu.VMEM((NIDX,), jnp.int32),
        pltpu.VMEM((N_LEVELS * F_PER_LEVEL, C), jnp.float32),
        pltpu.SemaphoreType.DMA,
    ],
    compiler_params=pltpu.CompilerParams(
        needs_layout_passes=False, use_tc_tiling_on_sc=False
    ),
)


def kernel(x, table):
    xf = x.reshape(-1)
    tb = table.astype(jnp.bfloat16)
    u0 = lax.bitcast_convert_type(tb[..., 0], jnp.uint16).astype(jnp.uint32)
    u1 = lax.bitcast_convert_type(tb[..., 1], jnp.uint16).astype(jnp.uint32)
    tf = lax.bitcast_convert_type(u0 | (u1 << 16), jnp.int32).reshape(-1)
    out = _call(xf, tf)
    return out.T


# speedup vs baseline: 4.6377x; 1.1277x over previous
from math import exp, log

import numpy as np
import jax
import jax.numpy as jnp
from jax import lax
from jax.experimental import pallas as pl
from jax.experimental.pallas import tpu as pltpu
from jax.experimental.pallas import tpu_sc as plsc

N_LEVELS = 16
F_PER_LEVEL = 2
LOG2_T = 19
T = 1 << LOG2_T
BASE_RES = 16
MAX_RES = 2048
SCALE = exp((log(MAX_RES) - log(BASE_RES)) / (N_LEVELS - 1))
RES = [int(np.floor(BASE_RES * (SCALE ** l))) for l in range(N_LEVELS)]
DENSE = [(r + 1) ** 3 <= T for r in RES]
P1 = np.int32(-1640531535)
P2 = np.int32(805459861)
HMASK = np.int32(T - 1)
CORNERS = [(i, j, k) for i in (0, 1) for j in (0, 1) for k in (0, 1)]

N_POINTS = 524288
NC, NS = 2, 16
NW = NC * NS
NPT = N_POINTS // NW
C = 512
G = C // 16
NCH = NPT // C
NIDX = 8 * C
HI16 = np.int32(-65536)


def _body(x_hbm, tab_hbm, out_hbm, xbuf, fracbuf, idxbuf, gath, outc, sem):
    cid = lax.axis_index("c")
    sid = lax.axis_index("s")
    wid = sid * NC + cid
    lane = lax.iota(jnp.int32, 16)
    lane3 = lane * 3

    def chunk_body(ch, carry):
        base = wid * NPT + ch * C
        pltpu.sync_copy(x_hbm.at[pl.ds(base * 3, C * 3)], xbuf)

        def make_p1(l):
            res = RES[l]
            resf = np.float32(res)
            b = l & 1

            def p1(g, c1):
                b16 = g * 16
                xi = b16 * 3 + lane3
                px = plsc.load_gather(xbuf, [xi])
                py = plsc.load_gather(xbuf, [xi + 1])
                pz = plsc.load_gather(xbuf, [xi + 2])
                posx = px * resf
                posy = py * resf
                posz = pz * resf
                ix = posx.astype(jnp.int32)
                iy = posy.astype(jnp.int32)
                iz = posz.astype(jnp.int32)
                fracbuf[0, pl.ds(b16, 16)] = posx - ix.astype(jnp.float32)
                fracbuf[1, pl.ds(b16, 16)] = posy - iy.astype(jnp.float32)
                fracbuf[2, pl.ds(b16, 16)] = posz - iz.astype(jnp.float32)
                for ci, (i, j, k) in enumerate(CORNERS):
                    cx = ix + i if i else ix
                    cy = iy + j if j else iy
                    cz = iz + k if k else iz
                    if DENSE[l]:
                        s = np.int32(res + 1)
                        idx = cx + cy * s + cz * np.int32((res + 1) * (res + 1))
                    else:
                        idx = (cx ^ (cy * P1) ^ (cz * P2)) & HMASK
                    idxbuf[b, pl.ds(ci * C + b16, 16)] = idx + np.int32(l * T)
                return c1

            return p1

        def make_p2(l):
            b = l & 1

            def p2(g, c2):
                b16 = g * 16
                fx = fracbuf[0, pl.ds(b16, 16)]
                fy = fracbuf[1, pl.ds(b16, 16)]
                fz = fracbuf[2, pl.ds(b16, 16)]
                gx = 1.0 - fx
                gy = 1.0 - fy
                gz = 1.0 - fz
                wyz = (gy * gz, gy * fz, fy * gz, fy * fz)
                acc0 = jnp.zeros((16,), jnp.float32)
                acc1 = jnp.zeros((16,), jnp.float32)
                for ci, (i, j, k) in enumerate(CORNERS):
                    v = gath[b, pl.ds(ci * C + b16, 16)]
                    f0 = plsc.bitcast(lax.shift_left(v, 16), jnp.float32)
                    f1 = plsc.bitcast(v & HI16, jnp.float32)
                    w = (fx if i else gx) * wyz[2 * j + k]
                    acc0 = acc0 + w * f0
                    acc1 = acc1 + w * f1
                outc[2 * l, pl.ds(b16, 16)] = acc0
                outc[2 * l + 1, pl.ds(b16, 16)] = acc1
                return c2

            return p2

        def fire(l):
            b = l & 1
            pltpu.async_copy(tab_hbm.at[idxbuf.at[b]], gath.at[b], sem)

        def wait(l):
            b = l & 1
            pltpu.make_async_copy(tab_hbm.at[idxbuf.at[b]], gath.at[b], sem).wait()

        lax.fori_loop(0, G, make_p1(0), 0, unroll=False)
        fire(0)
        for l in range(1, N_LEVELS + 1):
            if l < N_LEVELS:
                lax.fori_loop(0, G, make_p1(l), 0, unroll=False)
            wait(l - 1)
            if l < N_LEVELS:
                fire(l)
            lax.fori_loop(0, G, make_p2(l - 1), 0, unroll=False)
        pltpu.sync_copy(outc, out_hbm.at[:, pl.ds(base, C)])
        return carry

    lax.fori_loop(0, NCH, chunk_body, 0, unroll=False)


_mesh = plsc.VectorSubcoreMesh(
    core_axis_name="c", subcore_axis_name="s", num_cores=2, num_subcores=16
)

_call = pl.kernel(
    _body,
    out_type=jax.ShapeDtypeStruct((N_LEVELS * F_PER_LEVEL, N_POINTS), jnp.float32),
    mesh=_mesh,
    scratch_types=[
        pltpu.VMEM((3 * C,), jnp.float32),
        pltpu.VMEM((3, C), jnp.float32),
        pltpu.VMEM((2, NIDX), jnp.int32),
        pltpu.VMEM((2, NIDX), jnp.int32),
        pltpu.VMEM((N_LEVELS * F_PER_LEVEL, C), jnp.float32),
        pltpu.SemaphoreType.DMA,
    ],
    compiler_params=pltpu.CompilerParams(
        needs_layout_passes=False, use_tc_tiling_on_sc=False
    ),
)


def kernel(x, table):
    xf = x.reshape(-1)
    tb = table.astype(jnp.bfloat16)
    u0 = lax.bitcast_convert_type(tb[..., 0], jnp.uint16).astype(jnp.uint32)
    u1 = lax.bitcast_convert_type(tb[..., 1], jnp.uint16).astype(jnp.uint32)
    tf = lax.bitcast_convert_type(u0 | (u1 << 16), jnp.int32).reshape(-1)
    out = _call(xf, tf)
    return out.T
